# baseline (speedup 1.0000x reference)
import functools

import jax
import jax.numpy as jnp
from jax import lax
from jax.experimental import pallas as pl
from jax.experimental.pallas import tpu as pltpu
from jax.experimental.pallas import tpu_sc as plsc

N = 10000
N2 = 10240
D = 256
HALF = 128
E = 160000
EP = 163840
NC = 2
NS = 16
NPT = N2 // NS
R = 1000

_mesh = plsc.VectorSubcoreMesh(core_axis_name="c", subcore_axis_name="s")


_DEG_EDGES_PER_W = EP // (NC * NS)
_DEG_CHUNKS = _DEG_EDGES_PER_W // 128


@functools.partial(
    pl.kernel,
    out_type=(jax.ShapeDtypeStruct((N2, 128), jnp.float32),
              jax.ShapeDtypeStruct((N2, 128), jnp.float32)),
    mesh=_mesh,
    scratch_types=[
        pltpu.VMEM((128,), jnp.int32),
        pltpu.VMEM((128, 128), jnp.float32),
        pltpu.VMEM_SHARED((N2, 128), jnp.float32),
    ],
)
def _sc_deg(col_hbm, ones_hbm, zeros_hbm, d0_hbm, d1_hbm, idx_v, ones_v, deg_sp):
    c = lax.axis_index("c")
    s = lax.axis_index("s")
    w = c * NS + s
    base = w * _DEG_EDGES_PER_W
    pltpu.sync_copy(ones_hbm, ones_v)
    pltpu.sync_copy(zeros_hbm, deg_sp.at[pl.ds(s * NPT, NPT)])
    plsc.subcore_barrier()

    def step(j, carry):
        pltpu.sync_copy(col_hbm.at[pl.ds(base + 128 * j, 128)], idx_v)
        pltpu.sync_copy(ones_v, deg_sp.at[idx_v], add=True)
        return carry

    lax.fori_loop(0, _DEG_CHUNKS, step, 0)
    plsc.subcore_barrier()

    @pl.when(c == 0)
    def _():
        pltpu.sync_copy(deg_sp.at[pl.ds(s * NPT, NPT)],
                        d0_hbm.at[pl.ds(s * NPT, NPT)])

    @pl.when(c == 1)
    def _():
        pltpu.sync_copy(deg_sp.at[pl.ds(s * NPT, NPT)],
                        d1_hbm.at[pl.ds(s * NPT, NPT)])


_PROP_EDGES_PER_T = EP // NS
CH = 64
_PROP_CHUNKS = _PROP_EDGES_PER_T // CH


@functools.partial(
    pl.kernel,
    out_type=(jax.ShapeDtypeStruct((N2, HALF), jnp.float32),
              jax.ShapeDtypeStruct((N2, HALF), jnp.float32)),
    mesh=_mesh,
    scratch_types=(
        [pltpu.VMEM((_PROP_EDGES_PER_T,), jnp.int32)]
        + [pltpu.VMEM((CH,), jnp.int32) for _ in range(4)]
        + [pltpu.VMEM((CH, HALF), jnp.float32) for _ in range(4)]
        + [pltpu.VMEM_SHARED((N2, HALF), jnp.float32)]
        + [pltpu.SemaphoreType.DMA for _ in range(8)]
    ),
)
def _sc_prop(y0_hbm, y1_hbm, row_hbm, col_hbm, a0_hbm, a1_hbm,
             row_v, col_0, col_1, col_2, col_3,
             rows_0, rows_1, rows_2, rows_3, acc_sp,
             sg0, sg1, sg2, sg3, sc0, sc1, sc2, sc3):
    c = lax.axis_index("c")
    s = lax.axis_index("s")
    base = s * _PROP_EDGES_PER_T
    NB = 4
    cols = (col_0, col_1, col_2, col_3)
    rows = (rows_0, rows_1, rows_2, rows_3)
    gsems = (sg0, sg1, sg2, sg3)
    csems = (sc0, sc1, sc2, sc3)

    def run(y_hbm, a_hbm):
        pltpu.sync_copy(y_hbm.at[pl.ds(s * NPT, NPT)],
                        acc_sp.at[pl.ds(s * NPT, NPT)])
        pltpu.sync_copy(row_hbm.at[pl.ds(base, _PROP_EDGES_PER_T)], row_v)
        plsc.subcore_barrier()

        def fetch(k, b):
            pltpu.async_copy(y_hbm.at[row_v.at[pl.ds(CH * k, CH)]],
                             rows[b], gsems[b])
            pltpu.async_copy(col_hbm.at[pl.ds(base + CH * k, CH)],
                             cols[b], csems[b])

        def drain_scatter(k, b):
            pltpu.make_async_copy(y_hbm.at[row_v.at[pl.ds(CH * k, CH)]],
                                  rows[b], gsems[b]).wait()
            pltpu.make_async_copy(col_hbm.at[pl.ds(base + CH * k, CH)],
                                  cols[b], csems[b]).wait()
            pltpu.sync_copy(rows[b], acc_sp.at[cols[b]], add=True)

        for b in range(NB):
            fetch(b, b)

        def step(j, carry):
            for b in range(NB):
                drain_scatter(NB * j + b, b)
                fetch(NB * j + NB + b, b)
            return carry

        lax.fori_loop(0, _PROP_CHUNKS // NB - 1, step, 0)
        for b in range(NB):
            drain_scatter(_PROP_CHUNKS - NB + b, b)
        plsc.subcore_barrier()
        pltpu.sync_copy(acc_sp.at[pl.ds(s * NPT, NPT)],
                        a_hbm.at[pl.ds(s * NPT, NPT)])

    @pl.when(c == 0)
    def _():
        run(y0_hbm, a0_hbm)

    @pl.when(c == 1)
    def _():
        run(y1_hbm, a1_hbm)


def _deg_from(d0, d1):
    deg = jnp.sum(d0, axis=1, keepdims=True) + jnp.sum(d1, axis=1, keepdims=True)
    return deg * (1.0 / 128.0) + 1.0


@functools.partial(
    pl.pallas_call,
    out_shape=jax.ShapeDtypeStruct((N, D), jnp.float32),
    grid=(N // R,),
    in_specs=[
        pl.BlockSpec((R, D), lambda i: (i, 0)),
        pl.BlockSpec((D, D), lambda i: (0, 0)),
    ],
    out_specs=pl.BlockSpec((R, D), lambda i: (i, 0)),
)
def _tc_mm(h_ref, w_ref, x_ref):
    x_ref[...] = jnp.dot(h_ref[...], w_ref[...],
                         preferred_element_type=jnp.float32)


@functools.partial(
    pl.pallas_call,
    out_shape=(jax.ShapeDtypeStruct((N2, HALF), jnp.float32),
               jax.ShapeDtypeStruct((N2, HALF), jnp.float32),
               jax.ShapeDtypeStruct((N, 8), jnp.float32)),
    grid=(N // R,),
    in_specs=[
        pl.BlockSpec((R, D), lambda i: (i, 0)),
        pl.BlockSpec((R, 128), lambda i: (i, 0)),
        pl.BlockSpec((R, 128), lambda i: (i, 0)),
    ],
    out_specs=(pl.BlockSpec((R, HALF), lambda i: (i, 0)),
               pl.BlockSpec((R, HALF), lambda i: (i, 0)),
               pl.BlockSpec((R, 8), lambda i: (i, 0))),
)
def _tc_scale(x_ref, d0_ref, d1_ref, y0_ref, y1_ref, dinv_ref):
    dinv = lax.rsqrt(_deg_from(d0_ref[...], d1_ref[...]))
    y = x_ref[...] * dinv
    y0_ref[...] = y[:, :HALF]
    y1_ref[...] = y[:, HALF:]
    dinv_ref[...] = jnp.broadcast_to(dinv, (R, 8))


@functools.partial(
    pl.pallas_call,
    out_shape=jax.ShapeDtypeStruct((N, D), jnp.float32),
    grid=(2, N // R),
    in_specs=[
        pl.BlockSpec((R, HALF), lambda k, i: (i, 0)),
        pl.BlockSpec((R, HALF), lambda k, i: (i, 0)),
        pl.BlockSpec((R, 8), lambda k, i: (i, 0)),
        pl.BlockSpec((R, D), lambda k, i: (i, 0)),
        pl.BlockSpec((1, D), lambda k, i: (0, 0)),
        pl.BlockSpec((1, D), lambda k, i: (0, 0)),
        pl.BlockSpec((1, D), lambda k, i: (0, 0)),
    ],
    out_specs=pl.BlockSpec((R, D), lambda k, i: (i, 0)),
    scratch_shapes=[pltpu.VMEM((1, D), jnp.float32),
                    pltpu.VMEM((1, D), jnp.float32)],
)
def _tc_final(a0_ref, a1_ref, dinv_ref, h_ref, b_ref, g_ref, be_ref,
              out_ref, acc, accsq):
    k = pl.program_id(0)
    i = pl.program_id(1)
    dinv = dinv_ref[...][:, :1]
    pre = jnp.concatenate([a0_ref[...], a1_ref[...]], axis=1) * dinv + b_ref[...]

    @pl.when((k == 0) & (i == 0))
    def _():
        acc[...] = jnp.zeros_like(acc)
        accsq[...] = jnp.zeros_like(accsq)

    @pl.when(k == 0)
    def _():
        acc[...] += jnp.sum(pre, axis=0, keepdims=True)
        accsq[...] += jnp.sum(pre * pre, axis=0, keepdims=True)

    @pl.when(k == 1)
    def _():
        mean = acc[...] * (1.0 / N)
        var = accsq[...] * (1.0 / N) - mean * mean
        inv = lax.rsqrt(var + 1e-5)
        o = (pre - mean) * inv * g_ref[...] + be_ref[...]
        out_ref[...] = h_ref[...] + jnp.maximum(o, 0.0)


def kernel(h, edge_index, W, b, gamma, beta):
    row = edge_index[0].astype(jnp.int32)
    col = edge_index[1].astype(jnp.int32)
    pad = EP - E
    row_p = jnp.concatenate([row, jnp.zeros((pad,), jnp.int32)])
    col_p = jnp.concatenate([col, jnp.full((pad,), N, jnp.int32)])
    ones_rows = jnp.ones((128, 128), jnp.float32)
    zeros_init = jnp.zeros((NPT, 128), jnp.float32)

    d0, d1 = _sc_deg(col_p, ones_rows, zeros_init)
    x = _tc_mm(h, W)
    y0, y1, dinvw = _tc_scale(x, d0, d1)
    a0, a1 = _sc_prop(y0, y1, row_p, col_p)
    out = _tc_final(a0, a1, dinvw, h,
                    b.reshape(1, D), gamma.reshape(1, D), beta.reshape(1, D))
    return out


# speedup vs baseline: 1.0371x; 1.0371x over previous
import functools

import jax
import jax.numpy as jnp
from jax import lax
from jax.experimental import pallas as pl
from jax.experimental.pallas import tpu as pltpu
from jax.experimental.pallas import tpu_sc as plsc

N = 10000
N2 = 10240
D = 256
HALF = 128
E = 160000
EP = 163840
NC = 2
NS = 16
NPT = N2 // NS
R = 1000

_mesh = plsc.VectorSubcoreMesh(core_axis_name="c", subcore_axis_name="s")


_DEG_EDGES_PER_W = EP // (NC * NS)
_DEG_CHUNKS = _DEG_EDGES_PER_W // 128


@functools.partial(
    pl.kernel,
    out_type=(jax.ShapeDtypeStruct((N2, 128), jnp.float32),
              jax.ShapeDtypeStruct((N2, 128), jnp.float32)),
    mesh=_mesh,
    scratch_types=[
        pltpu.VMEM((128,), jnp.int32),
        pltpu.VMEM((128, 128), jnp.float32),
        pltpu.VMEM_SHARED((N2, 128), jnp.float32),
    ],
)
def _sc_deg(col_hbm, ones_hbm, zeros_hbm, d0_hbm, d1_hbm, idx_v, ones_v, deg_sp):
    c = lax.axis_index("c")
    s = lax.axis_index("s")
    w = c * NS + s
    base = w * _DEG_EDGES_PER_W
    pltpu.sync_copy(ones_hbm, ones_v)
    pltpu.sync_copy(zeros_hbm, deg_sp.at[pl.ds(s * NPT, NPT)])
    plsc.subcore_barrier()

    def step(j, carry):
        pltpu.sync_copy(col_hbm.at[pl.ds(base + 128 * j, 128)], idx_v)
        pltpu.sync_copy(ones_v, deg_sp.at[idx_v], add=True)
        return carry

    lax.fori_loop(0, _DEG_CHUNKS, step, 0)
    plsc.subcore_barrier()

    @pl.when(c == 0)
    def _():
        pltpu.sync_copy(deg_sp.at[pl.ds(s * NPT, NPT)],
                        d0_hbm.at[pl.ds(s * NPT, NPT)])

    @pl.when(c == 1)
    def _():
        pltpu.sync_copy(deg_sp.at[pl.ds(s * NPT, NPT)],
                        d1_hbm.at[pl.ds(s * NPT, NPT)])


_PROP_EDGES_PER_T = EP // NS
CH = 64
_PROP_CHUNKS = _PROP_EDGES_PER_T // CH


@functools.partial(
    pl.kernel,
    out_type=(jax.ShapeDtypeStruct((N2, HALF), jnp.float32),
              jax.ShapeDtypeStruct((N2, HALF), jnp.float32)),
    mesh=_mesh,
    scratch_types=(
        [pltpu.VMEM((_PROP_EDGES_PER_T,), jnp.int32)]
        + [pltpu.VMEM((CH,), jnp.int32) for _ in range(4)]
        + [pltpu.VMEM((CH, HALF), jnp.float32) for _ in range(4)]
        + [pltpu.VMEM_SHARED((N2, HALF), jnp.float32)]
        + [pltpu.SemaphoreType.DMA for _ in range(8)]
    ),
)
def _sc_prop(y0_hbm, y1_hbm, row_hbm, col_hbm, a0_hbm, a1_hbm,
             row_v, col_0, col_1, col_2, col_3,
             rows_0, rows_1, rows_2, rows_3, acc_sp,
             sg0, sg1, sg2, sg3, sc0, sc1, sc2, sc3):
    c = lax.axis_index("c")
    s = lax.axis_index("s")
    base = s * _PROP_EDGES_PER_T
    NB = 4
    cols = (col_0, col_1, col_2, col_3)
    rows = (rows_0, rows_1, rows_2, rows_3)
    gsems = (sg0, sg1, sg2, sg3)
    csems = (sc0, sc1, sc2, sc3)

    def run(y_hbm, a_hbm):
        pltpu.sync_copy(y_hbm.at[pl.ds(s * NPT, NPT)],
                        acc_sp.at[pl.ds(s * NPT, NPT)])
        pltpu.sync_copy(row_hbm.at[pl.ds(base, _PROP_EDGES_PER_T)], row_v)
        plsc.subcore_barrier()

        def fetch(k, b):
            pltpu.async_copy(y_hbm.at[row_v.at[pl.ds(CH * k, CH)]],
                             rows[b], gsems[b])
            pltpu.async_copy(col_hbm.at[pl.ds(base + CH * k, CH)],
                             cols[b], csems[b])

        def drain_scatter(k, b):
            pltpu.make_async_copy(y_hbm.at[row_v.at[pl.ds(CH * k, CH)]],
                                  rows[b], gsems[b]).wait()
            pltpu.make_async_copy(col_hbm.at[pl.ds(base + CH * k, CH)],
                                  cols[b], csems[b]).wait()
            pltpu.sync_copy(rows[b], acc_sp.at[cols[b]], add=True)

        for b in range(NB):
            fetch(b, b)

        def step(j, carry):
            for b in range(NB):
                drain_scatter(NB * j + b, b)
                fetch(NB * j + NB + b, b)
            return carry

        lax.fori_loop(0, _PROP_CHUNKS // NB - 1, step, 0)
        for b in range(NB):
            drain_scatter(_PROP_CHUNKS - NB + b, b)
        plsc.subcore_barrier()
        pltpu.sync_copy(acc_sp.at[pl.ds(s * NPT, NPT)],
                        a_hbm.at[pl.ds(s * NPT, NPT)])

    @pl.when(c == 0)
    def _():
        run(y0_hbm, a0_hbm)

    @pl.when(c == 1)
    def _():
        run(y1_hbm, a1_hbm)


def _deg_from(d0, d1):
    deg = jnp.sum(d0, axis=1, keepdims=True) + jnp.sum(d1, axis=1, keepdims=True)
    return deg * (1.0 / 128.0) + 1.0


@functools.partial(
    pl.pallas_call,
    out_shape=(jax.ShapeDtypeStruct((N2, HALF), jnp.float32),
               jax.ShapeDtypeStruct((N2, HALF), jnp.float32),
               jax.ShapeDtypeStruct((N, 8), jnp.float32)),
    grid=(N // R,),
    in_specs=[
        pl.BlockSpec((R, D), lambda i: (i, 0)),
        pl.BlockSpec((D, D), lambda i: (0, 0)),
        pl.BlockSpec((R, 128), lambda i: (i, 0)),
        pl.BlockSpec((R, 128), lambda i: (i, 0)),
    ],
    out_specs=(pl.BlockSpec((R, HALF), lambda i: (i, 0)),
               pl.BlockSpec((R, HALF), lambda i: (i, 0)),
               pl.BlockSpec((R, 8), lambda i: (i, 0))),
)
def _tc_scale(h_ref, w_ref, d0_ref, d1_ref, y0_ref, y1_ref, dinv_ref):
    dinv = lax.rsqrt(_deg_from(d0_ref[...], d1_ref[...]))
    y = jnp.dot(h_ref[...], w_ref[...],
                preferred_element_type=jnp.float32) * dinv
    y0_ref[...] = y[:, :HALF]
    y1_ref[...] = y[:, HALF:]
    dinv_ref[...] = jnp.broadcast_to(dinv, (R, 8))


@functools.partial(
    pl.pallas_call,
    out_shape=jax.ShapeDtypeStruct((N, D), jnp.float32),
    grid=(2, N // R),
    in_specs=[
        pl.BlockSpec((R, HALF), lambda k, i: (i, 0)),
        pl.BlockSpec((R, HALF), lambda k, i: (i, 0)),
        pl.BlockSpec((R, 8), lambda k, i: (i, 0)),
        pl.BlockSpec((R, D), lambda k, i: (i, 0)),
        pl.BlockSpec((1, D), lambda k, i: (0, 0)),
        pl.BlockSpec((1, D), lambda k, i: (0, 0)),
        pl.BlockSpec((1, D), lambda k, i: (0, 0)),
    ],
    out_specs=pl.BlockSpec((R, D), lambda k, i: (i, 0)),
    scratch_shapes=[pltpu.VMEM((1, D), jnp.float32),
                    pltpu.VMEM((1, D), jnp.float32)],
)
def _tc_final(a0_ref, a1_ref, dinv_ref, h_ref, b_ref, g_ref, be_ref,
              out_ref, acc, accsq):
    k = pl.program_id(0)
    i = pl.program_id(1)
    dinv = dinv_ref[...][:, :1]
    pre = jnp.concatenate([a0_ref[...], a1_ref[...]], axis=1) * dinv + b_ref[...]

    @pl.when((k == 0) & (i == 0))
    def _():
        acc[...] = jnp.zeros_like(acc)
        accsq[...] = jnp.zeros_like(accsq)

    @pl.when(k == 0)
    def _():
        acc[...] += jnp.sum(pre, axis=0, keepdims=True)
        accsq[...] += jnp.sum(pre * pre, axis=0, keepdims=True)

    @pl.when(k == 1)
    def _():
        mean = acc[...] * (1.0 / N)
        var = accsq[...] * (1.0 / N) - mean * mean
        inv = lax.rsqrt(var + 1e-5)
        o = (pre - mean) * inv * g_ref[...] + be_ref[...]
        out_ref[...] = h_ref[...] + jnp.maximum(o, 0.0)


def kernel(h, edge_index, W, b, gamma, beta):
    row = edge_index[0].astype(jnp.int32)
    col = edge_index[1].astype(jnp.int32)
    pad = EP - E
    row_p = jnp.concatenate([row, jnp.zeros((pad,), jnp.int32)])
    col_p = jnp.concatenate([col, jnp.full((pad,), N, jnp.int32)])
    ones_rows = jnp.ones((128, 128), jnp.float32)
    zeros_init = jnp.zeros((NPT, 128), jnp.float32)

    d0, d1 = _sc_deg(col_p, ones_rows, zeros_init)
    y0, y1, dinvw = _tc_scale(h, W, d0, d1)
    a0, a1 = _sc_prop(y0, y1, row_p, col_p)
    out = _tc_final(a0, a1, dinvw, h,
                    b.reshape(1, D), gamma.reshape(1, D), beta.reshape(1, D))
    return out


# speedup vs baseline: 1.0747x; 1.0363x over previous
import functools

import jax
import jax.numpy as jnp
from jax import lax
from jax.experimental import pallas as pl
from jax.experimental.pallas import tpu as pltpu
from jax.experimental.pallas import tpu_sc as plsc

N = 10000
N2 = 10240
D = 256
HALF = 128
E = 160000
EP = 163840
NC = 2
NS = 16
NPT = N2 // NS
R = 1000

_mesh = plsc.VectorSubcoreMesh(core_axis_name="c", subcore_axis_name="s")


_DEG_EDGES_PER_W = EP // (NC * NS)
_DEG_CHUNKS = _DEG_EDGES_PER_W // 128


@functools.partial(
    pl.kernel,
    out_type=(jax.ShapeDtypeStruct((N2, 128), jnp.float32),
              jax.ShapeDtypeStruct((N2, 128), jnp.float32)),
    mesh=_mesh,
    scratch_types=[
        pltpu.VMEM((128,), jnp.int32),
        pltpu.VMEM((128,), jnp.int32),
        pltpu.VMEM((128, 128), jnp.float32),
        pltpu.VMEM_SHARED((N2, 128), jnp.float32),
        pltpu.SemaphoreType.DMA,
        pltpu.SemaphoreType.DMA,
    ],
)
def _sc_deg(col_hbm, ones_hbm, zeros_hbm, d0_hbm, d1_hbm, idx_a, idx_b, ones_v,
            deg_sp, sem_a, sem_b):
    c = lax.axis_index("c")
    s = lax.axis_index("s")
    w = c * NS + s
    base = w * _DEG_EDGES_PER_W
    idxs = (idx_a, idx_b)
    sems = (sem_a, sem_b)
    pltpu.sync_copy(ones_hbm, ones_v)

    def fetch(k, p):
        pltpu.async_copy(col_hbm.at[pl.ds(base + 128 * k, 128)],
                         idxs[p], sems[p])

    def drain_scatter(k, p):
        pltpu.make_async_copy(col_hbm.at[pl.ds(base + 128 * k, 128)],
                              idxs[p], sems[p]).wait()
        pltpu.sync_copy(ones_v, deg_sp.at[idxs[p]], add=True)

    fetch(0, 0)
    fetch(1, 1)
    pltpu.sync_copy(zeros_hbm, deg_sp.at[pl.ds(s * NPT, NPT)])
    plsc.subcore_barrier()

    def step(j, carry):
        drain_scatter(2 * j, 0)
        fetch(2 * j + 2, 0)
        drain_scatter(2 * j + 1, 1)
        fetch(2 * j + 3, 1)
        return carry

    lax.fori_loop(0, _DEG_CHUNKS // 2 - 1, step, 0)
    drain_scatter(_DEG_CHUNKS - 2, 0)
    drain_scatter(_DEG_CHUNKS - 1, 1)
    plsc.subcore_barrier()

    @pl.when(c == 0)
    def _():
        pltpu.sync_copy(deg_sp.at[pl.ds(s * NPT, NPT)],
                        d0_hbm.at[pl.ds(s * NPT, NPT)])

    @pl.when(c == 1)
    def _():
        pltpu.sync_copy(deg_sp.at[pl.ds(s * NPT, NPT)],
                        d1_hbm.at[pl.ds(s * NPT, NPT)])


_PROP_EDGES_PER_T = EP // NS
CH = 64
_PROP_CHUNKS = _PROP_EDGES_PER_T // CH


@functools.partial(
    pl.kernel,
    out_type=(jax.ShapeDtypeStruct((N2, HALF), jnp.float32),
              jax.ShapeDtypeStruct((N2, HALF), jnp.float32)),
    mesh=_mesh,
    scratch_types=(
        [pltpu.VMEM((_PROP_EDGES_PER_T,), jnp.int32)]
        + [pltpu.VMEM((CH,), jnp.int32) for _ in range(4)]
        + [pltpu.VMEM((CH, HALF), jnp.float32) for _ in range(4)]
        + [pltpu.VMEM_SHARED((N2, HALF), jnp.float32)]
        + [pltpu.SemaphoreType.DMA for _ in range(8)]
    ),
)
def _sc_prop(y0_hbm, y1_hbm, row_hbm, col_hbm, a0_hbm, a1_hbm,
             row_v, col_0, col_1, col_2, col_3,
             rows_0, rows_1, rows_2, rows_3, acc_sp,
             sg0, sg1, sg2, sg3, sc0, sc1, sc2, sc3):
    c = lax.axis_index("c")
    s = lax.axis_index("s")
    base = s * _PROP_EDGES_PER_T
    NB = 4
    cols = (col_0, col_1, col_2, col_3)
    rows = (rows_0, rows_1, rows_2, rows_3)
    gsems = (sg0, sg1, sg2, sg3)
    csems = (sc0, sc1, sc2, sc3)

    def run(y_hbm, a_hbm):
        pltpu.sync_copy(y_hbm.at[pl.ds(s * NPT, NPT)],
                        acc_sp.at[pl.ds(s * NPT, NPT)])
        pltpu.sync_copy(row_hbm.at[pl.ds(base, _PROP_EDGES_PER_T)], row_v)
        plsc.subcore_barrier()

        def fetch(k, b):
            pltpu.async_copy(y_hbm.at[row_v.at[pl.ds(CH * k, CH)]],
                             rows[b], gsems[b])
            pltpu.async_copy(col_hbm.at[pl.ds(base + CH * k, CH)],
                             cols[b], csems[b])

        def drain_scatter(k, b):
            pltpu.make_async_copy(y_hbm.at[row_v.at[pl.ds(CH * k, CH)]],
                                  rows[b], gsems[b]).wait()
            pltpu.make_async_copy(col_hbm.at[pl.ds(base + CH * k, CH)],
                                  cols[b], csems[b]).wait()
            pltpu.sync_copy(rows[b], acc_sp.at[cols[b]], add=True)

        for b in range(NB):
            fetch(b, b)

        def step(j, carry):
            for b in range(NB):
                drain_scatter(NB * j + b, b)
                fetch(NB * j + NB + b, b)
            return carry

        lax.fori_loop(0, _PROP_CHUNKS // NB - 1, step, 0)
        for b in range(NB):
            drain_scatter(_PROP_CHUNKS - NB + b, b)
        plsc.subcore_barrier()
        pltpu.sync_copy(acc_sp.at[pl.ds(s * NPT, NPT)],
                        a_hbm.at[pl.ds(s * NPT, NPT)])

    @pl.when(c == 0)
    def _():
        run(y0_hbm, a0_hbm)

    @pl.when(c == 1)
    def _():
        run(y1_hbm, a1_hbm)


def _deg_from(d0, d1):
    deg = jnp.sum(d0, axis=1, keepdims=True) + jnp.sum(d1, axis=1, keepdims=True)
    return deg * (1.0 / 128.0) + 1.0


@functools.partial(
    pl.pallas_call,
    out_shape=(jax.ShapeDtypeStruct((N2, HALF), jnp.float32),
               jax.ShapeDtypeStruct((N2, HALF), jnp.float32),
               jax.ShapeDtypeStruct((N, 8), jnp.float32)),
    grid=(N // R,),
    in_specs=[
        pl.BlockSpec((R, D), lambda i: (i, 0)),
        pl.BlockSpec((D, D), lambda i: (0, 0)),
        pl.BlockSpec((R, 128), lambda i: (i, 0)),
        pl.BlockSpec((R, 128), lambda i: (i, 0)),
    ],
    out_specs=(pl.BlockSpec((R, HALF), lambda i: (i, 0)),
               pl.BlockSpec((R, HALF), lambda i: (i, 0)),
               pl.BlockSpec((R, 8), lambda i: (i, 0))),
)
def _tc_scale(h_ref, w_ref, d0_ref, d1_ref, y0_ref, y1_ref, dinv_ref):
    dinv = lax.rsqrt(_deg_from(d0_ref[...], d1_ref[...]))
    y = jnp.dot(h_ref[...], w_ref[...],
                preferred_element_type=jnp.float32) * dinv
    y0_ref[...] = y[:, :HALF]
    y1_ref[...] = y[:, HALF:]
    dinv_ref[...] = jnp.broadcast_to(dinv, (R, 8))


@functools.partial(
    pl.pallas_call,
    out_shape=jax.ShapeDtypeStruct((N, D), jnp.float32),
    grid=(2, N // R),
    in_specs=[
        pl.BlockSpec((R, HALF), lambda k, i: (i, 0)),
        pl.BlockSpec((R, HALF), lambda k, i: (i, 0)),
        pl.BlockSpec((R, 8), lambda k, i: (i, 0)),
        pl.BlockSpec((R, D), lambda k, i: (i * k, 0)),
        pl.BlockSpec((1, D), lambda k, i: (0, 0)),
        pl.BlockSpec((1, D), lambda k, i: (0, 0)),
        pl.BlockSpec((1, D), lambda k, i: (0, 0)),
    ],
    out_specs=pl.BlockSpec((R, D), lambda k, i: (i * k, 0)),
    scratch_shapes=[pltpu.VMEM((1, D), jnp.float32),
                    pltpu.VMEM((1, D), jnp.float32)],
)
def _tc_final(a0_ref, a1_ref, dinv_ref, h_ref, b_ref, g_ref, be_ref,
              out_ref, acc, accsq):
    k = pl.program_id(0)
    i = pl.program_id(1)
    dinv = dinv_ref[...][:, :1]
    pre = jnp.concatenate([a0_ref[...], a1_ref[...]], axis=1) * dinv + b_ref[...]

    @pl.when((k == 0) & (i == 0))
    def _():
        acc[...] = jnp.zeros_like(acc)
        accsq[...] = jnp.zeros_like(accsq)

    @pl.when(k == 0)
    def _():
        acc[...] += jnp.sum(pre, axis=0, keepdims=True)
        accsq[...] += jnp.sum(pre * pre, axis=0, keepdims=True)

    @pl.when(k == 1)
    def _():
        mean = acc[...] * (1.0 / N)
        var = accsq[...] * (1.0 / N) - mean * mean
        inv = lax.rsqrt(var + 1e-5)
        o = (pre - mean) * inv * g_ref[...] + be_ref[...]
        out_ref[...] = h_ref[...] + jnp.maximum(o, 0.0)


def kernel(h, edge_index, W, b, gamma, beta):
    row = edge_index[0].astype(jnp.int32)
    col = edge_index[1].astype(jnp.int32)
    pad = EP - E
    row_p = jnp.concatenate([row, jnp.zeros((pad,), jnp.int32)])
    col_p = jnp.concatenate([col, jnp.full((pad,), N, jnp.int32)])
    ones_rows = jnp.ones((128, 128), jnp.float32)
    zeros_init = jnp.zeros((NPT, 128), jnp.float32)

    d0, d1 = _sc_deg(col_p, ones_rows, zeros_init)
    y0, y1, dinvw = _tc_scale(h, W, d0, d1)
    a0, a1 = _sc_prop(y0, y1, row_p, col_p)
    out = _tc_final(a0, a1, dinvw, h,
                    b.reshape(1, D), gamma.reshape(1, D), beta.reshape(1, D))
    return out


# speedup vs baseline: 1.1105x; 1.0332x over previous
import functools

import jax
import jax.numpy as jnp
from jax import lax
from jax.experimental import pallas as pl
from jax.experimental.pallas import tpu as pltpu
from jax.experimental.pallas import tpu_sc as plsc

N = 10000
N2 = 10240
D = 256
HALF = 128
E = 160000
EP = 163840
NC = 2
NS = 16
NPT = N2 // NS
R = 2000

_mesh = plsc.VectorSubcoreMesh(core_axis_name="c", subcore_axis_name="s")


_DEG_EDGES_PER_W = EP // (NC * NS)
_DEG_CHUNKS = _DEG_EDGES_PER_W // 128


@functools.partial(
    pl.kernel,
    out_type=(jax.ShapeDtypeStruct((N2, 128), jnp.float32),
              jax.ShapeDtypeStruct((N2, 128), jnp.float32)),
    mesh=_mesh,
    scratch_types=[
        pltpu.VMEM((128,), jnp.int32),
        pltpu.VMEM((128,), jnp.int32),
        pltpu.VMEM((128, 128), jnp.float32),
        pltpu.VMEM_SHARED((N2, 128), jnp.float32),
        pltpu.SemaphoreType.DMA,
        pltpu.SemaphoreType.DMA,
    ],
)
def _sc_deg(col_hbm, ones_hbm, zeros_hbm, d0_hbm, d1_hbm, idx_a, idx_b, ones_v,
            deg_sp, sem_a, sem_b):
    c = lax.axis_index("c")
    s = lax.axis_index("s")
    w = c * NS + s
    base = w * _DEG_EDGES_PER_W
    idxs = (idx_a, idx_b)
    sems = (sem_a, sem_b)
    pltpu.sync_copy(ones_hbm, ones_v)

    def fetch(k, p):
        pltpu.async_copy(col_hbm.at[pl.ds(base + 128 * k, 128)],
                         idxs[p], sems[p])

    def drain_scatter(k, p):
        pltpu.make_async_copy(col_hbm.at[pl.ds(base + 128 * k, 128)],
                              idxs[p], sems[p]).wait()
        pltpu.sync_copy(ones_v, deg_sp.at[idxs[p]], add=True)

    fetch(0, 0)
    fetch(1, 1)
    pltpu.sync_copy(zeros_hbm, deg_sp.at[pl.ds(s * NPT, NPT)])
    plsc.subcore_barrier()

    def step(j, carry):
        drain_scatter(2 * j, 0)
        fetch(2 * j + 2, 0)
        drain_scatter(2 * j + 1, 1)
        fetch(2 * j + 3, 1)
        return carry

    lax.fori_loop(0, _DEG_CHUNKS // 2 - 1, step, 0)
    drain_scatter(_DEG_CHUNKS - 2, 0)
    drain_scatter(_DEG_CHUNKS - 1, 1)
    plsc.subcore_barrier()

    @pl.when(c == 0)
    def _():
        pltpu.sync_copy(deg_sp.at[pl.ds(s * NPT, NPT)],
                        d0_hbm.at[pl.ds(s * NPT, NPT)])

    @pl.when(c == 1)
    def _():
        pltpu.sync_copy(deg_sp.at[pl.ds(s * NPT, NPT)],
                        d1_hbm.at[pl.ds(s * NPT, NPT)])


_PROP_EDGES_PER_T = EP // NS
CH = 64
_PROP_CHUNKS = _PROP_EDGES_PER_T // CH


@functools.partial(
    pl.kernel,
    out_type=(jax.ShapeDtypeStruct((N2, HALF), jnp.float32),
              jax.ShapeDtypeStruct((N2, HALF), jnp.float32)),
    mesh=_mesh,
    scratch_types=(
        [pltpu.VMEM((_PROP_EDGES_PER_T,), jnp.int32)]
        + [pltpu.VMEM((CH,), jnp.int32) for _ in range(4)]
        + [pltpu.VMEM((CH, HALF), jnp.float32) for _ in range(4)]
        + [pltpu.VMEM_SHARED((N2, HALF), jnp.float32)]
        + [pltpu.SemaphoreType.DMA for _ in range(8)]
    ),
)
def _sc_prop(y0_hbm, y1_hbm, row_hbm, col_hbm, a0_hbm, a1_hbm,
             row_v, col_0, col_1, col_2, col_3,
             rows_0, rows_1, rows_2, rows_3, acc_sp,
             sg0, sg1, sg2, sg3, sc0, sc1, sc2, sc3):
    c = lax.axis_index("c")
    s = lax.axis_index("s")
    base = s * _PROP_EDGES_PER_T
    NB = 4
    cols = (col_0, col_1, col_2, col_3)
    rows = (rows_0, rows_1, rows_2, rows_3)
    gsems = (sg0, sg1, sg2, sg3)
    csems = (sc0, sc1, sc2, sc3)

    def run(y_hbm, a_hbm):
        pltpu.sync_copy(y_hbm.at[pl.ds(s * NPT, NPT)],
                        acc_sp.at[pl.ds(s * NPT, NPT)])
        pltpu.sync_copy(row_hbm.at[pl.ds(base, _PROP_EDGES_PER_T)], row_v)
        plsc.subcore_barrier()

        def fetch(k, b):
            pltpu.async_copy(y_hbm.at[row_v.at[pl.ds(CH * k, CH)]],
                             rows[b], gsems[b])
            pltpu.async_copy(col_hbm.at[pl.ds(base + CH * k, CH)],
                             cols[b], csems[b])

        def drain_scatter(k, b):
            pltpu.make_async_copy(y_hbm.at[row_v.at[pl.ds(CH * k, CH)]],
                                  rows[b], gsems[b]).wait()
            pltpu.make_async_copy(col_hbm.at[pl.ds(base + CH * k, CH)],
                                  cols[b], csems[b]).wait()
            pltpu.sync_copy(rows[b], acc_sp.at[cols[b]], add=True)

        for b in range(NB):
            fetch(b, b)

        def step(j, carry):
            for b in range(NB):
                drain_scatter(NB * j + b, b)
                fetch(NB * j + NB + b, b)
            return carry

        lax.fori_loop(0, _PROP_CHUNKS // NB - 1, step, 0)
        for b in range(NB):
            drain_scatter(_PROP_CHUNKS - NB + b, b)
        plsc.subcore_barrier()
        pltpu.sync_copy(acc_sp.at[pl.ds(s * NPT, NPT)],
                        a_hbm.at[pl.ds(s * NPT, NPT)])

    @pl.when(c == 0)
    def _():
        run(y0_hbm, a0_hbm)

    @pl.when(c == 1)
    def _():
        run(y1_hbm, a1_hbm)


def _deg_from(d0, d1):
    deg = jnp.sum(d0, axis=1, keepdims=True) + jnp.sum(d1, axis=1, keepdims=True)
    return deg * (1.0 / 128.0) + 1.0


@functools.partial(
    pl.pallas_call,
    out_shape=(jax.ShapeDtypeStruct((N2, HALF), jnp.float32),
               jax.ShapeDtypeStruct((N2, HALF), jnp.float32),
               jax.ShapeDtypeStruct((N, 8), jnp.float32)),
    grid=(N // R,),
    in_specs=[
        pl.BlockSpec((R, D), lambda i: (i, 0)),
        pl.BlockSpec((D, D), lambda i: (0, 0)),
        pl.BlockSpec((R, 128), lambda i: (i, 0)),
        pl.BlockSpec((R, 128), lambda i: (i, 0)),
    ],
    out_specs=(pl.BlockSpec((R, HALF), lambda i: (i, 0)),
               pl.BlockSpec((R, HALF), lambda i: (i, 0)),
               pl.BlockSpec((R, 8), lambda i: (i, 0))),
)
def _tc_scale(h_ref, w_ref, d0_ref, d1_ref, y0_ref, y1_ref, dinv_ref):
    dinv = lax.rsqrt(_deg_from(d0_ref[...], d1_ref[...]))
    y = jnp.dot(h_ref[...], w_ref[...],
                preferred_element_type=jnp.float32) * dinv
    y0_ref[...] = y[:, :HALF]
    y1_ref[...] = y[:, HALF:]
    dinv_ref[...] = jnp.broadcast_to(dinv, (R, 8))


@functools.partial(
    pl.pallas_call,
    out_shape=jax.ShapeDtypeStruct((N, D), jnp.float32),
    grid=(2, N // R),
    in_specs=[
        pl.BlockSpec((R, HALF), lambda k, i: (i, 0)),
        pl.BlockSpec((R, HALF), lambda k, i: (i, 0)),
        pl.BlockSpec((R, 8), lambda k, i: (i, 0)),
        pl.BlockSpec((R, D), lambda k, i: (i * k, 0)),
        pl.BlockSpec((1, D), lambda k, i: (0, 0)),
        pl.BlockSpec((1, D), lambda k, i: (0, 0)),
        pl.BlockSpec((1, D), lambda k, i: (0, 0)),
    ],
    out_specs=pl.BlockSpec((R, D), lambda k, i: (i * k, 0)),
    scratch_shapes=[pltpu.VMEM((1, D), jnp.float32),
                    pltpu.VMEM((1, D), jnp.float32)],
)
def _tc_final(a0_ref, a1_ref, dinv_ref, h_ref, b_ref, g_ref, be_ref,
              out_ref, acc, accsq):
    k = pl.program_id(0)
    i = pl.program_id(1)
    dinv = dinv_ref[...][:, :1]
    pre = jnp.concatenate([a0_ref[...], a1_ref[...]], axis=1) * dinv + b_ref[...]

    @pl.when((k == 0) & (i == 0))
    def _():
        acc[...] = jnp.zeros_like(acc)
        accsq[...] = jnp.zeros_like(accsq)

    @pl.when(k == 0)
    def _():
        acc[...] += jnp.sum(pre, axis=0, keepdims=True)
        accsq[...] += jnp.sum(pre * pre, axis=0, keepdims=True)

    @pl.when(k == 1)
    def _():
        mean = acc[...] * (1.0 / N)
        var = accsq[...] * (1.0 / N) - mean * mean
        inv = lax.rsqrt(var + 1e-5)
        o = (pre - mean) * inv * g_ref[...] + be_ref[...]
        out_ref[...] = h_ref[...] + jnp.maximum(o, 0.0)


def kernel(h, edge_index, W, b, gamma, beta):
    row = edge_index[0].astype(jnp.int32)
    col = edge_index[1].astype(jnp.int32)
    pad = EP - E
    row_p = jnp.concatenate([row, jnp.zeros((pad,), jnp.int32)])
    col_p = jnp.concatenate([col, jnp.full((pad,), N, jnp.int32)])
    ones_rows = jnp.ones((128, 128), jnp.float32)
    zeros_init = jnp.zeros((NPT, 128), jnp.float32)

    d0, d1 = _sc_deg(col_p, ones_rows, zeros_init)
    y0, y1, dinvw = _tc_scale(h, W, d0, d1)
    a0, a1 = _sc_prop(y0, y1, row_p, col_p)
    out = _tc_final(a0, a1, dinvw, h,
                    b.reshape(1, D), gamma.reshape(1, D), beta.reshape(1, D))
    return out
